# baseline (speedup 1.0000x reference)
import numpy as np
import jax
import jax.numpy as jnp
from jax.experimental import pallas as pl
from jax.experimental.pallas import tpu as pltpu

TILE_H = 8


def _tap_table(lens):
    r = lens // 2
    ys, xs = np.meshgrid(np.arange(lens) - r, np.arange(lens) - r,
                         indexing='ij')
    d = np.sqrt(ys.astype(np.float64) ** 2 + xs.astype(np.float64) ** 2)
    d32 = d.astype(np.float32)
    mask = d32 <= r + 1e-6
    return [(dy, dx, float(d32[dy, dx]))
            for dy in range(lens) for dx in range(lens) if mask[dy, dx]]


def _prep_body(le_ref, disp_ref, u_ref):
    b = pl.program_id(0)
    u_ref[...] = jnp.abs(disp_ref[...]) * le_ref[b, 0]


WIN = TILE_H + 16


def _main_body(taps, u_ref, p_ref, out_ref):
    t = pl.program_id(1)
    r0 = pl.multiple_of(t * TILE_H, TILE_H)
    u_win = u_ref[0, pl.ds(r0, WIN), :]
    p_win = p_ref[0, :, pl.ds(r0, WIN), :]
    dxs = sorted({dx for _, dx, _ in taps})
    u_sh = {dx: u_win[:, dx:dx + 512] for dx in dxs}
    a_sh = {dx: p_win[3, :, dx:dx + 512] for dx in dxs}
    rgb_sh = {dx: p_win[:3, :, dx:dx + 512] for dx in dxs}
    acc = jnp.zeros((3, TILE_H, 512), jnp.float32)
    wsum = jnp.zeros((TILE_H, 512), jnp.float32)
    for dy, dx, dval in taps:
        usl = u_sh[dx][dy:dy + TILE_H, :]
        cov = jnp.minimum(jnp.maximum((usl - dval) + 0.5, 0.0), 1.0)
        w = cov * a_sh[dx][dy:dy + TILE_H, :]
        acc = acc + w[None, :, :] * rgb_sh[dx][:, dy:dy + TILE_H, :]
        wsum = wsum + w
    out_ref[...] = (acc / (wsum + 1e-6)[None, :, :])[None]


def kernel(x, lens_effects, diskernel, lens_mask):
    b, c, h, w = x.shape
    lens = diskernel.shape[0]
    pad = lens // 2
    taps = _tap_table(lens)

    disp = x[:, 4]
    u = pl.pallas_call(
        _prep_body,
        grid=(b,),
        in_specs=[
            pl.BlockSpec(memory_space=pltpu.SMEM),
            pl.BlockSpec((1, h, w), lambda i: (i, 0, 0)),
        ],
        out_specs=pl.BlockSpec((1, h, w), lambda i: (i, 0, 0)),
        out_shape=jax.ShapeDtypeStruct((b, h, w), jnp.float32),
    )(lens_effects, disp)

    hp = h + 2 * pad
    wp = w + 2 * pad
    hp8 = ((hp + 7) // 8) * 8
    wp128 = ((wp + 127) // 128) * 128

    u_pad = jnp.pad(u, ((0, 0), (pad, pad), (pad, pad)), mode='edge')
    u_pad = jnp.pad(u_pad, ((0, 0), (0, hp8 - hp), (0, wp128 - wp)))
    rgba = x[:, :4]
    p_pad = jnp.pad(rgba, ((0, 0), (0, 0), (pad, pad), (pad, pad)),
                    mode='edge')
    p_pad = jnp.pad(p_pad, ((0, 0), (0, 0), (0, hp8 - hp), (0, wp128 - wp)))

    out = pl.pallas_call(
        lambda u_ref, p_ref, o_ref: _main_body(taps, u_ref, p_ref, o_ref),
        grid=(b, h // TILE_H),
        in_specs=[
            pl.BlockSpec((1, hp8, wp128), lambda i, t: (i, 0, 0)),
            pl.BlockSpec((1, 4, hp8, wp128), lambda i, t: (i, 0, 0, 0)),
        ],
        out_specs=pl.BlockSpec((1, 3, TILE_H, w), lambda i, t: (i, 0, t, 0)),
        out_shape=jax.ShapeDtypeStruct((b, 3, h, w), jnp.float32),
        compiler_params=pltpu.CompilerParams(
            dimension_semantics=("parallel", "parallel")),
    )(u_pad, p_pad)
    return out


# speedup vs baseline: 1.4572x; 1.4572x over previous
import numpy as np
import jax
import jax.numpy as jnp
from jax.experimental import pallas as pl
from jax.experimental.pallas import tpu as pltpu

TILE_H = 8


def _tap_table(lens):
    r = lens // 2
    ys, xs = np.meshgrid(np.arange(lens) - r, np.arange(lens) - r,
                         indexing='ij')
    d = np.sqrt(ys.astype(np.float64) ** 2 + xs.astype(np.float64) ** 2)
    d32 = d.astype(np.float32)
    mask = d32 <= r + 1e-6
    return [(dy, dx, float(d32[dy, dx]))
            for dy in range(lens) for dx in range(lens) if mask[dy, dx]]


def _prep_body(le_ref, disp_ref, u_ref):
    b = pl.program_id(0)
    u_ref[...] = jnp.abs(disp_ref[...]) * le_ref[b, 0]


WIN = TILE_H + 16


def _main_body(taps, u_ref, p_ref, out_ref):
    t = pl.program_id(1)
    r0 = pl.multiple_of(t * TILE_H, TILE_H)
    u_win = u_ref[0, pl.ds(r0, WIN), :]
    p_win = p_ref[0, :, pl.ds(r0, WIN), :]

    wtab = {}
    for dy, dx, dval in taps:
        usl = u_win[dy:dy + TILE_H, :]
        cov = jnp.minimum(jnp.maximum((usl - dval) + 0.5, 0.0), 1.0)
        wtab[(dy, dx)] = cov * p_win[3, dy:dy + TILE_H, :]

    wsum = jnp.zeros((TILE_H, 512), jnp.float32)
    for dy, dx, dval in taps:
        wsum = wsum + wtab[(dy, dx)][:, dx:dx + 512]

    cols = {}
    for dy, dx, dval in taps:
        cols.setdefault(dx, []).append((dy, dval))
    acc = [jnp.zeros((TILE_H, 512), jnp.float32) for _ in range(3)]
    for dx in sorted(cols):
        cacc = [jnp.zeros((TILE_H, 640), jnp.float32) for _ in range(3)]
        for dy, dval in cols[dx]:
            w = wtab[(dy, dx)]
            cacc = [a + w * p_win[c, dy:dy + TILE_H, :]
                    for a, c in zip(cacc, range(3))]
        acc = [a + c[:, dx:dx + 512] for a, c in zip(acc, cacc)]
    out_ref[...] = jnp.stack(acc)[None] / (wsum + 1e-6)[None, None, :, :]


def kernel(x, lens_effects, diskernel, lens_mask):
    b, c, h, w = x.shape
    lens = diskernel.shape[0]
    pad = lens // 2
    taps = _tap_table(lens)

    disp = x[:, 4]
    u = pl.pallas_call(
        _prep_body,
        grid=(b,),
        in_specs=[
            pl.BlockSpec(memory_space=pltpu.SMEM),
            pl.BlockSpec((1, h, w), lambda i: (i, 0, 0)),
        ],
        out_specs=pl.BlockSpec((1, h, w), lambda i: (i, 0, 0)),
        out_shape=jax.ShapeDtypeStruct((b, h, w), jnp.float32),
    )(lens_effects, disp)

    hp = h + 2 * pad
    wp = w + 2 * pad
    hp8 = ((hp + 7) // 8) * 8
    wp128 = ((wp + 127) // 128) * 128

    u_pad = jnp.pad(u, ((0, 0), (pad, pad), (pad, pad)), mode='edge')
    u_pad = jnp.pad(u_pad, ((0, 0), (0, hp8 - hp), (0, wp128 - wp)))
    rgba = x[:, :4]
    p_pad = jnp.pad(rgba, ((0, 0), (0, 0), (pad, pad), (pad, pad)),
                    mode='edge')
    p_pad = jnp.pad(p_pad, ((0, 0), (0, 0), (0, hp8 - hp), (0, wp128 - wp)))

    out = pl.pallas_call(
        lambda u_ref, p_ref, o_ref: _main_body(taps, u_ref, p_ref, o_ref),
        grid=(b, h // TILE_H),
        in_specs=[
            pl.BlockSpec((1, hp8, wp128), lambda i, t: (i, 0, 0)),
            pl.BlockSpec((1, 4, hp8, wp128), lambda i, t: (i, 0, 0, 0)),
        ],
        out_specs=pl.BlockSpec((1, 3, TILE_H, w), lambda i, t: (i, 0, t, 0)),
        out_shape=jax.ShapeDtypeStruct((b, 3, h, w), jnp.float32),
    )(u_pad, p_pad)
    return out


# speedup vs baseline: 1.6503x; 1.1325x over previous
import numpy as np
import jax
import jax.numpy as jnp
from jax.experimental import pallas as pl
from jax.experimental.pallas import tpu as pltpu

TILE_H = 16


def _tap_table(lens):
    r = lens // 2
    ys, xs = np.meshgrid(np.arange(lens) - r, np.arange(lens) - r,
                         indexing='ij')
    d = np.sqrt(ys.astype(np.float64) ** 2 + xs.astype(np.float64) ** 2)
    d32 = d.astype(np.float32)
    mask = d32 <= r + 1e-6
    return [(dy, dx, float(d32[dy, dx]))
            for dy in range(lens) for dx in range(lens) if mask[dy, dx]]


def _prep_body(le_ref, disp_ref, u_ref):
    b = pl.program_id(0)
    u_ref[...] = jnp.abs(disp_ref[...]) * le_ref[b, 0]


WIN = TILE_H + 16


def _main_body(taps, u_ref, p_ref, out_ref):
    t = pl.program_id(1)
    r0 = pl.multiple_of(t * TILE_H, TILE_H)
    u_win = u_ref[0, pl.ds(r0, WIN), :]
    p_win = p_ref[0, :, pl.ds(r0, WIN), :]

    wtab = {}
    for dy, dx, dval in taps:
        if (dy, dval) not in wtab:
            usl = u_win[dy:dy + TILE_H, :]
            cov = jnp.minimum(jnp.maximum((usl - dval) + 0.5, 0.0), 1.0)
            wtab[(dy, dval)] = cov * p_win[3, dy:dy + TILE_H, :]

    wsum = jnp.zeros((TILE_H, 512), jnp.float32)
    for dy, dx, dval in taps:
        wsum = wsum + wtab[(dy, dval)][:, dx:dx + 512]

    cols = {}
    for dy, dx, dval in taps:
        cols.setdefault(dx, []).append((dy, dval))
    acc = [jnp.zeros((TILE_H, 512), jnp.float32) for _ in range(3)]
    for dx in sorted(cols):
        cacc = [jnp.zeros((TILE_H, 640), jnp.float32) for _ in range(3)]
        for dy, dval in cols[dx]:
            w = wtab[(dy, dval)]
            cacc = [a + w * p_win[c, dy:dy + TILE_H, :]
                    for a, c in zip(cacc, range(3))]
        acc = [a + c[:, dx:dx + 512] for a, c in zip(acc, cacc)]
    out_ref[...] = jnp.stack(acc)[None] / (wsum + 1e-6)[None, None, :, :]


def kernel(x, lens_effects, diskernel, lens_mask):
    b, c, h, w = x.shape
    lens = diskernel.shape[0]
    pad = lens // 2
    taps = _tap_table(lens)

    disp = x[:, 4]
    u = pl.pallas_call(
        _prep_body,
        grid=(b,),
        in_specs=[
            pl.BlockSpec(memory_space=pltpu.SMEM),
            pl.BlockSpec((1, h, w), lambda i: (i, 0, 0)),
        ],
        out_specs=pl.BlockSpec((1, h, w), lambda i: (i, 0, 0)),
        out_shape=jax.ShapeDtypeStruct((b, h, w), jnp.float32),
    )(lens_effects, disp)

    hp = h + 2 * pad
    wp = w + 2 * pad
    hp8 = ((hp + 7) // 8) * 8
    wp128 = ((wp + 127) // 128) * 128

    u_pad = jnp.pad(u, ((0, 0), (pad, pad), (pad, pad)), mode='edge')
    u_pad = jnp.pad(u_pad, ((0, 0), (0, hp8 - hp), (0, wp128 - wp)))
    rgba = x[:, :4]
    p_pad = jnp.pad(rgba, ((0, 0), (0, 0), (pad, pad), (pad, pad)),
                    mode='edge')
    p_pad = jnp.pad(p_pad, ((0, 0), (0, 0), (0, hp8 - hp), (0, wp128 - wp)))

    out = pl.pallas_call(
        lambda u_ref, p_ref, o_ref: _main_body(taps, u_ref, p_ref, o_ref),
        grid=(b, h // TILE_H),
        in_specs=[
            pl.BlockSpec((1, hp8, wp128), lambda i, t: (i, 0, 0)),
            pl.BlockSpec((1, 4, hp8, wp128), lambda i, t: (i, 0, 0, 0)),
        ],
        out_specs=pl.BlockSpec((1, 3, TILE_H, w), lambda i, t: (i, 0, t, 0)),
        out_shape=jax.ShapeDtypeStruct((b, 3, h, w), jnp.float32),
    )(u_pad, p_pad)
    return out


# speedup vs baseline: 1.7590x; 1.0658x over previous
import numpy as np
import jax
import jax.numpy as jnp
from jax.experimental import pallas as pl
from jax.experimental.pallas import tpu as pltpu

TILE_H = 32


def _tap_table(lens):
    r = lens // 2
    ys, xs = np.meshgrid(np.arange(lens) - r, np.arange(lens) - r,
                         indexing='ij')
    d = np.sqrt(ys.astype(np.float64) ** 2 + xs.astype(np.float64) ** 2)
    d32 = d.astype(np.float32)
    mask = d32 <= r + 1e-6
    return [(dy, dx, float(d32[dy, dx]))
            for dy in range(lens) for dx in range(lens) if mask[dy, dx]]


def _prep_body(le_ref, disp_ref, u_ref):
    b = pl.program_id(0)
    u_ref[...] = jnp.abs(disp_ref[...]) * le_ref[b, 0]


WIN = TILE_H + 16


def _main_body(taps, u_ref, p_ref, out_ref):
    t = pl.program_id(1)
    r0 = pl.multiple_of(t * TILE_H, TILE_H)
    u_win = u_ref[0, pl.ds(r0, WIN), :]
    p_win = p_ref[0, :, pl.ds(r0, WIN), :]

    wtab = {}
    for dy, dx, dval in taps:
        if (dy, dval) not in wtab:
            usl = u_win[dy:dy + TILE_H, :]
            cov = jnp.minimum(jnp.maximum((usl - dval) + 0.5, 0.0), 1.0)
            wtab[(dy, dval)] = cov * p_win[3, dy:dy + TILE_H, :]

    wsum = jnp.zeros((TILE_H, 512), jnp.float32)
    for dy, dx, dval in taps:
        wsum = wsum + wtab[(dy, dval)][:, dx:dx + 512]

    cols = {}
    for dy, dx, dval in taps:
        cols.setdefault(dx, []).append((dy, dval))
    acc = [jnp.zeros((TILE_H, 512), jnp.float32) for _ in range(3)]
    for dx in sorted(cols):
        cacc = [jnp.zeros((TILE_H, 640), jnp.float32) for _ in range(3)]
        for dy, dval in cols[dx]:
            w = wtab[(dy, dval)]
            cacc = [a + w * p_win[c, dy:dy + TILE_H, :]
                    for a, c in zip(cacc, range(3))]
        acc = [a + c[:, dx:dx + 512] for a, c in zip(acc, cacc)]
    out_ref[...] = jnp.stack(acc)[None] / (wsum + 1e-6)[None, None, :, :]


def kernel(x, lens_effects, diskernel, lens_mask):
    b, c, h, w = x.shape
    lens = diskernel.shape[0]
    pad = lens // 2
    taps = _tap_table(lens)

    disp = x[:, 4]
    u = pl.pallas_call(
        _prep_body,
        grid=(b,),
        in_specs=[
            pl.BlockSpec(memory_space=pltpu.SMEM),
            pl.BlockSpec((1, h, w), lambda i: (i, 0, 0)),
        ],
        out_specs=pl.BlockSpec((1, h, w), lambda i: (i, 0, 0)),
        out_shape=jax.ShapeDtypeStruct((b, h, w), jnp.float32),
    )(lens_effects, disp)

    hp = h + 2 * pad
    wp = w + 2 * pad
    hp8 = ((hp + 7) // 8) * 8
    wp128 = ((wp + 127) // 128) * 128

    u_pad = jnp.pad(u, ((0, 0), (pad, pad), (pad, pad)), mode='edge')
    u_pad = jnp.pad(u_pad, ((0, 0), (0, hp8 - hp), (0, wp128 - wp)))
    rgba = x[:, :4]
    p_pad = jnp.pad(rgba, ((0, 0), (0, 0), (pad, pad), (pad, pad)),
                    mode='edge')
    p_pad = jnp.pad(p_pad, ((0, 0), (0, 0), (0, hp8 - hp), (0, wp128 - wp)))

    out = pl.pallas_call(
        lambda u_ref, p_ref, o_ref: _main_body(taps, u_ref, p_ref, o_ref),
        grid=(b, h // TILE_H),
        in_specs=[
            pl.BlockSpec((1, hp8, wp128), lambda i, t: (i, 0, 0)),
            pl.BlockSpec((1, 4, hp8, wp128), lambda i, t: (i, 0, 0, 0)),
        ],
        out_specs=pl.BlockSpec((1, 3, TILE_H, w), lambda i, t: (i, 0, t, 0)),
        out_shape=jax.ShapeDtypeStruct((b, 3, h, w), jnp.float32),
    )(u_pad, p_pad)
    return out


# speedup vs baseline: 1.7982x; 1.0223x over previous
import numpy as np
import jax
import jax.numpy as jnp
from jax.experimental import pallas as pl
from jax.experimental.pallas import tpu as pltpu

TILE_H = 64


def _tap_table(lens):
    r = lens // 2
    ys, xs = np.meshgrid(np.arange(lens) - r, np.arange(lens) - r,
                         indexing='ij')
    d = np.sqrt(ys.astype(np.float64) ** 2 + xs.astype(np.float64) ** 2)
    d32 = d.astype(np.float32)
    mask = d32 <= r + 1e-6
    return [(dy, dx, float(d32[dy, dx]))
            for dy in range(lens) for dx in range(lens) if mask[dy, dx]]


def _prep_body(le_ref, disp_ref, u_ref):
    b = pl.program_id(0)
    u_ref[...] = jnp.abs(disp_ref[...]) * le_ref[b, 0]


WIN = TILE_H + 16


def _main_body(taps, u_ref, p_ref, out_ref):
    t = pl.program_id(1)
    r0 = pl.multiple_of(t * TILE_H, TILE_H)
    u_win = u_ref[0, pl.ds(r0, WIN), :]
    p_win = p_ref[0, :, pl.ds(r0, WIN), :]

    wtab = {}
    for dy, dx, dval in taps:
        if (dy, dval) not in wtab:
            usl = u_win[dy:dy + TILE_H, :]
            cov = jnp.minimum(jnp.maximum((usl - dval) + 0.5, 0.0), 1.0)
            wtab[(dy, dval)] = cov * p_win[3, dy:dy + TILE_H, :]

    wsum = jnp.zeros((TILE_H, 512), jnp.float32)
    for dy, dx, dval in taps:
        wsum = wsum + wtab[(dy, dval)][:, dx:dx + 512]

    cols = {}
    for dy, dx, dval in taps:
        cols.setdefault(dx, []).append((dy, dval))
    acc = [jnp.zeros((TILE_H, 512), jnp.float32) for _ in range(3)]
    for dx in sorted(cols):
        cacc = [jnp.zeros((TILE_H, 640), jnp.float32) for _ in range(3)]
        for dy, dval in cols[dx]:
            w = wtab[(dy, dval)]
            cacc = [a + w * p_win[c, dy:dy + TILE_H, :]
                    for a, c in zip(cacc, range(3))]
        acc = [a + c[:, dx:dx + 512] for a, c in zip(acc, cacc)]
    out_ref[...] = jnp.stack(acc)[None] / (wsum + 1e-6)[None, None, :, :]


def kernel(x, lens_effects, diskernel, lens_mask):
    b, c, h, w = x.shape
    lens = diskernel.shape[0]
    pad = lens // 2
    taps = _tap_table(lens)

    disp = x[:, 4]
    u = pl.pallas_call(
        _prep_body,
        grid=(b,),
        in_specs=[
            pl.BlockSpec(memory_space=pltpu.SMEM),
            pl.BlockSpec((1, h, w), lambda i: (i, 0, 0)),
        ],
        out_specs=pl.BlockSpec((1, h, w), lambda i: (i, 0, 0)),
        out_shape=jax.ShapeDtypeStruct((b, h, w), jnp.float32),
    )(lens_effects, disp)

    hp = h + 2 * pad
    wp = w + 2 * pad
    hp8 = ((hp + 7) // 8) * 8
    wp128 = ((wp + 127) // 128) * 128

    u_pad = jnp.pad(u, ((0, 0), (pad, pad), (pad, pad)), mode='edge')
    u_pad = jnp.pad(u_pad, ((0, 0), (0, hp8 - hp), (0, wp128 - wp)))
    rgba = x[:, :4]
    p_pad = jnp.pad(rgba, ((0, 0), (0, 0), (pad, pad), (pad, pad)),
                    mode='edge')
    p_pad = jnp.pad(p_pad, ((0, 0), (0, 0), (0, hp8 - hp), (0, wp128 - wp)))

    out = pl.pallas_call(
        lambda u_ref, p_ref, o_ref: _main_body(taps, u_ref, p_ref, o_ref),
        grid=(b, h // TILE_H),
        in_specs=[
            pl.BlockSpec((1, hp8, wp128), lambda i, t: (i, 0, 0)),
            pl.BlockSpec((1, 4, hp8, wp128), lambda i, t: (i, 0, 0, 0)),
        ],
        out_specs=pl.BlockSpec((1, 3, TILE_H, w), lambda i, t: (i, 0, t, 0)),
        out_shape=jax.ShapeDtypeStruct((b, 3, h, w), jnp.float32),
    )(u_pad, p_pad)
    return out
